# baseline (speedup 1.0000x reference)
import functools

import jax
import jax.numpy as jnp
from jax import lax
from jax.experimental import pallas as pl
from jax.experimental.pallas import tpu as pltpu
from jax.experimental.pallas import tpu_sc as plsc

NUM_NODES = 1000000
NUM_CAND = 65536
GLOBAL_DIM = 128
LOCAL_DIM = 128
HID = 32

NC = 2
NS = 16
NW = NC * NS

CHUNK = 31256
CHUNK_LAST = NUM_NODES - (NW - 1) * CHUNK

H = 2
CAND_H = NUM_CAND // H
CAND_CH = 4096
N_CH_H = CAND_H // CAND_CH
ROWS_CH = CAND_CH // 128

CPT_H = CAND_H // NS
GROWS_H = CPT_H // 128

GPT = NUM_CAND // NW
GPROWS = GPT // 128

BR = 8192
GRID_H = CAND_H // BR


def _sc_gather_body(gs_hbm, idx_hbm, g_hbm, idxg, gbuf, gsem):
    cid = lax.axis_index("c")
    sid = lax.axis_index("s")
    wid = sid * NC + cid
    gbase = wid * GPT
    pltpu.sync_copy(idx_hbm.at[pl.ds(gbase, GPT)], idxg)

    @pl.loop(0, GPROWS)
    def _(j):
        pltpu.async_copy(gs_hbm.at[idxg.at[pl.ds(j * 128, 128)]],
                         gbuf.at[j], gsem)

    @pl.loop(0, GPROWS)
    def _(j):
        pltpu.make_async_copy(gs_hbm.at[idxg.at[pl.ds(j * 128, 128)]],
                              gbuf.at[j], gsem).wait()

    pltpu.sync_copy(gbuf, g_hbm.at[pl.ds(wid * GPROWS, GPROWS)])


_sc_gather = functools.partial(
    pl.kernel,
    out_type=jax.ShapeDtypeStruct((NUM_CAND // 128, 128), jnp.float32),
    mesh=plsc.VectorSubcoreMesh(core_axis_name="c", subcore_axis_name="s",
                                num_cores=NC, num_subcores=NS),
    scratch_types=[
        pltpu.VMEM((GPT,), jnp.int32),
        pltpu.VMEM((GPROWS, 128), jnp.float32),
        pltpu.SemaphoreType.DMA,
    ],
    compiler_params=pltpu.CompilerParams(needs_layout_passes=False),
)(_sc_gather_body)


def _ffn_body(g_ref, l_ref, w1a_ref, w1b_ref, b1_ref, w2_ref, b2_ref, o_ref):
    dn = (((0,), (1,)), ((), ()))
    ht = lax.dot_general(w1a_ref[...], g_ref[...], dn,
                         preferred_element_type=jnp.float32)
    ht = ht + lax.dot_general(w1b_ref[...], l_ref[...], dn,
                              preferred_element_type=jnp.float32)
    ht = jnp.maximum(ht + b1_ref[...], 0.0)
    st = lax.dot_general(w2_ref[...], ht, (((0,), (0,)), ((), ())),
                         preferred_element_type=jnp.float32)
    o_ref[...] = jax.nn.sigmoid(st + b2_ref[...]).reshape(1, 1, BR)


def _compute_sigma_half(global_emb_h, local_emb_h, w1a, w1b, b1c, W2, b2c):
    out = pl.pallas_call(
        _ffn_body,
        grid=(GRID_H,),
        in_specs=[
            pl.BlockSpec((BR, GLOBAL_DIM), lambda i: (i, 0)),
            pl.BlockSpec((BR, LOCAL_DIM), lambda i: (i, 0)),
            pl.BlockSpec((GLOBAL_DIM, HID), lambda i: (0, 0)),
            pl.BlockSpec((LOCAL_DIM, HID), lambda i: (0, 0)),
            pl.BlockSpec((HID, 1), lambda i: (0, 0)),
            pl.BlockSpec((HID, 1), lambda i: (0, 0)),
            pl.BlockSpec((1, 1), lambda i: (0, 0)),
        ],
        out_specs=pl.BlockSpec((1, 1, BR), lambda i: (i, 0, 0)),
        out_shape=jax.ShapeDtypeStruct((GRID_H, 1, BR), jnp.float32),
    )(global_emb_h, local_emb_h, w1a, w1b, b1c, W2, b2c)
    return out.reshape(CAND_H)


def _sc_body(src_hbm, idx_hbm, sig_hbm, loc_hbm, g_hbm, out_hbm,
             fused_sh, vals_dst, sigb, locb, gbuf, fusedb,
             idxsb, fuseds, gsem, vsem, isem0, isem1, fsem0, fsem1):
    cid = lax.axis_index("c")
    sid = lax.axis_index("s")
    wid = sid * NC + cid
    base = wid * CHUNK
    is_last = wid == NW - 1
    cw = jnp.where(is_last, CHUNK_LAST, CHUNK).astype(jnp.int32)
    cwu = cw.astype(jnp.uint32)
    isems = (isem0, isem1)
    fsems = (fsem0, fsem1)

    @pl.when(jnp.logical_not(is_last))
    def _():
        pltpu.async_copy(src_hbm.at[pl.ds(base, CHUNK)], vals_dst, vsem)

    @pl.when(is_last)
    def _():
        pltpu.async_copy(src_hbm.at[pl.ds(base, CHUNK_LAST)],
                         vals_dst.at[pl.ds(0, CHUNK_LAST)], vsem)

    abase = sid * CPT_H
    pltpu.async_copy(g_hbm.at[pl.ds(sid * GROWS_H, GROWS_H)], gbuf, gsem)
    pltpu.sync_copy(sig_hbm.at[pl.ds(abase, CPT_H)], sigb)
    pltpu.sync_copy(loc_hbm.at[pl.ds(abase, CPT_H)], locb)
    pltpu.make_async_copy(g_hbm.at[pl.ds(sid * GROWS_H, GROWS_H)], gbuf,
                          gsem).wait()

    def fuse_row(j, carry):
        svs = [sigb[pl.ds(j * 128 + k * 16, 16)] for k in range(8)]
        lvs = [locb[pl.ds(j * 128 + k * 16, 16)] for k in range(8)]
        gvs = [gbuf[j, pl.ds(k * 16, 16)] for k in range(8)]
        for k in range(8):
            fusedb[pl.ds(j * 128 + k * 16, 16)] = (
                svs[k] * gvs[k] + (1.0 - svs[k]) * lvs[k])
        return carry

    lax.fori_loop(0, GROWS_H, fuse_row, 0)
    pltpu.sync_copy(fusedb, fused_sh.at[pl.ds(abase, CPT_H)])

    pltpu.async_copy(idx_hbm.at[pl.ds(0, CAND_CH)], idxsb.at[0], isems[0])

    plsc.subcore_barrier()

    pltpu.async_copy(fused_sh.at[pl.ds(0, CAND_CH)], fuseds.at[0], fsems[0])

    @pl.when(jnp.logical_not(is_last))
    def _():
        pltpu.make_async_copy(src_hbm.at[pl.ds(base, CHUNK)], vals_dst,
                              vsem).wait()

    @pl.when(is_last)
    def _():
        pltpu.make_async_copy(src_hbm.at[pl.ds(base, CHUNK_LAST)],
                              vals_dst.at[pl.ds(0, CHUNK_LAST)], vsem).wait()

    for c in range(N_CH_H):
        s = c & 1
        if c + 1 < N_CH_H:
            ns = 1 - s
            pltpu.async_copy(idx_hbm.at[pl.ds((c + 1) * CAND_CH, CAND_CH)],
                             idxsb.at[ns], isems[ns])
            pltpu.async_copy(fused_sh.at[pl.ds((c + 1) * CAND_CH, CAND_CH)],
                             fuseds.at[ns], fsems[ns])
        pltpu.make_async_copy(idx_hbm.at[pl.ds(c * CAND_CH, CAND_CH)],
                              idxsb.at[s], isems[s]).wait()
        pltpu.make_async_copy(fused_sh.at[pl.ds(c * CAND_CH, CAND_CH)],
                              fuseds.at[s], fsems[s]).wait()

        def row_body(j, carry, s=s):
            ivs = [idxsb[s, pl.ds(j * 128 + k * 16, 16)] for k in range(8)]
            fvs = [fuseds[s, pl.ds(j * 128 + k * 16, 16)] for k in range(8)]
            rels = [plsc.bitcast(iv - base, jnp.uint32) for iv in ivs]
            ms = [r < cwu for r in rels]
            relcs = [plsc.bitcast(jnp.minimum(r, jnp.uint32(CHUNK - 1)),
                                  jnp.int32) for r in rels]
            for k in range(8):
                plsc.store_scatter(vals_dst, [relcs[k]], fvs[k], mask=ms[k])
            return carry

        lax.fori_loop(0, ROWS_CH, row_body, 0)

    @pl.when(jnp.logical_not(is_last))
    def _():
        pltpu.sync_copy(vals_dst, out_hbm.at[pl.ds(base, CHUNK)])

    @pl.when(is_last)
    def _():
        pltpu.sync_copy(vals_dst.at[pl.ds(0, CHUNK_LAST)],
                        out_hbm.at[pl.ds(base, CHUNK_LAST)])


_sc_scatter = functools.partial(
    pl.kernel,
    out_type=jax.ShapeDtypeStruct((NUM_NODES,), jnp.float32),
    mesh=plsc.VectorSubcoreMesh(core_axis_name="c", subcore_axis_name="s",
                                num_cores=NC, num_subcores=NS),
    scratch_types=[
        pltpu.VMEM_SHARED((CAND_H,), jnp.float32),
        pltpu.VMEM((CHUNK,), jnp.float32),
        pltpu.VMEM((CPT_H,), jnp.float32),
        pltpu.VMEM((CPT_H,), jnp.float32),
        pltpu.VMEM((GROWS_H, 128), jnp.float32),
        pltpu.VMEM((CPT_H,), jnp.float32),
        pltpu.VMEM((2, CAND_CH), jnp.int32),
        pltpu.VMEM((2, CAND_CH), jnp.float32),
        pltpu.SemaphoreType.DMA,
        pltpu.SemaphoreType.DMA,
        pltpu.SemaphoreType.DMA,
        pltpu.SemaphoreType.DMA,
        pltpu.SemaphoreType.DMA,
        pltpu.SemaphoreType.DMA,
    ],
    compiler_params=pltpu.CompilerParams(needs_layout_passes=False),
)(_sc_body)


def kernel(global_emb, local_emb, global_scores, local_scores,
           candidate_indices, W1, b1, W2, b2):
    idx = candidate_indices.astype(jnp.int32)
    g2d = _sc_gather(global_scores, idx)

    w1a = W1[:GLOBAL_DIM]
    w1b = W1[GLOBAL_DIM:]
    b1c = b1.reshape(HID, 1)
    b2c = b2.reshape(1, 1)

    sigmas = []
    cur = global_scores
    for h in range(H):
        lo = h * CAND_H
        sig_h = _compute_sigma_half(
            global_emb[lo:lo + CAND_H], local_emb[lo:lo + CAND_H],
            w1a, w1b, b1c, W2, b2c)
        sigmas.append(sig_h)
        cur = _sc_scatter(cur, idx[lo:lo + CAND_H], sig_h,
                          local_scores[lo:lo + CAND_H],
                          g2d[h * (CAND_H // 128):(h + 1) * (CAND_H // 128)])

    sigma = jnp.concatenate(sigmas)
    return (cur, sigma)


# speedup vs baseline: 1.6997x; 1.6997x over previous
import functools

import jax
import jax.numpy as jnp
from jax import lax
from jax.experimental import pallas as pl
from jax.experimental.pallas import tpu as pltpu
from jax.experimental.pallas import tpu_sc as plsc

NUM_NODES = 1000000
NUM_CAND = 65536
GLOBAL_DIM = 128
LOCAL_DIM = 128
HID = 32

NC = 2
NS = 16
NW = NC * NS

CHUNK = 31256
CHUNK_LAST = NUM_NODES - (NW - 1) * CHUNK

CAND_CH = 4096
N_CH = NUM_CAND // CAND_CH

BR = 16384
GRID = NUM_CAND // BR


def _ffn_body(g_ref, l_ref, w1a_ref, w1b_ref, b1_ref, w2_ref, b2_ref, o_ref):
    dn = (((0,), (1,)), ((), ()))
    ht = lax.dot_general(w1a_ref[...], g_ref[...], dn,
                         preferred_element_type=jnp.float32)
    ht = ht + lax.dot_general(w1b_ref[...], l_ref[...], dn,
                              preferred_element_type=jnp.float32)
    ht = jnp.maximum(ht + b1_ref[...], 0.0)
    st = lax.dot_general(w2_ref[...], ht, (((0,), (0,)), ((), ())),
                         preferred_element_type=jnp.float32)
    o_ref[...] = jax.nn.sigmoid(st + b2_ref[...]).reshape(1, 1, BR)


def _compute_sigma(global_emb, local_emb, W1, b1, W2, b2):
    w1a = W1[:GLOBAL_DIM]
    w1b = W1[GLOBAL_DIM:]
    b1c = b1.reshape(HID, 1)
    b2c = b2.reshape(1, 1)
    out = pl.pallas_call(
        _ffn_body,
        grid=(GRID,),
        in_specs=[
            pl.BlockSpec((BR, GLOBAL_DIM), lambda i: (i, 0)),
            pl.BlockSpec((BR, LOCAL_DIM), lambda i: (i, 0)),
            pl.BlockSpec((GLOBAL_DIM, HID), lambda i: (0, 0)),
            pl.BlockSpec((LOCAL_DIM, HID), lambda i: (0, 0)),
            pl.BlockSpec((HID, 1), lambda i: (0, 0)),
            pl.BlockSpec((HID, 1), lambda i: (0, 0)),
            pl.BlockSpec((1, 1), lambda i: (0, 0)),
        ],
        out_specs=pl.BlockSpec((1, 1, BR), lambda i: (i, 0, 0)),
        out_shape=jax.ShapeDtypeStruct((GRID, 1, BR), jnp.float32),
    )(global_emb, local_emb, w1a, w1b, b1c, W2, b2c)
    return out.reshape(NUM_CAND)


CPT = NUM_CAND // NS
GROWS = CPT // 128
ROWS_CH = CAND_CH // 128

GPT = NUM_CAND // NW
GPROWS = GPT // 128


def _sc_gather_body(gs_hbm, idx_hbm, g_hbm, idxg, gbuf, gsem):
    cid = lax.axis_index("c")
    sid = lax.axis_index("s")
    wid = sid * NC + cid
    gbase = wid * GPT
    pltpu.sync_copy(idx_hbm.at[pl.ds(gbase, GPT)], idxg)

    @pl.loop(0, GPROWS)
    def _(j):
        pltpu.async_copy(gs_hbm.at[idxg.at[pl.ds(j * 128, 128)]],
                         gbuf.at[j], gsem)

    @pl.loop(0, GPROWS)
    def _(j):
        pltpu.make_async_copy(gs_hbm.at[idxg.at[pl.ds(j * 128, 128)]],
                              gbuf.at[j], gsem).wait()

    pltpu.sync_copy(gbuf, g_hbm.at[pl.ds(wid * GPROWS, GPROWS)])


_sc_gather = functools.partial(
    pl.kernel,
    out_type=jax.ShapeDtypeStruct((NUM_CAND // 128, 128), jnp.float32),
    mesh=plsc.VectorSubcoreMesh(core_axis_name="c", subcore_axis_name="s",
                                num_cores=NC, num_subcores=NS),
    scratch_types=[
        pltpu.VMEM((GPT,), jnp.int32),
        pltpu.VMEM((GPROWS, 128), jnp.float32),
        pltpu.SemaphoreType.DMA,
    ],
    compiler_params=pltpu.CompilerParams(needs_layout_passes=False),
)(_sc_gather_body)


def _sc_body(gs_hbm, idx_hbm, sig_hbm, loc_hbm, g_hbm, out_hbm,
             fused_sh, vals_dst, sigb, locb, gbuf, fusedb,
             idxsb, fuseds, gsem, vsem, isem0, isem1, fsem0, fsem1):
    cid = lax.axis_index("c")
    sid = lax.axis_index("s")
    wid = sid * NC + cid
    base = wid * CHUNK
    is_last = wid == NW - 1
    cw = jnp.where(is_last, CHUNK_LAST, CHUNK).astype(jnp.int32)
    cwu = cw.astype(jnp.uint32)
    isems = (isem0, isem1)
    fsems = (fsem0, fsem1)

    @pl.when(jnp.logical_not(is_last))
    def _():
        pltpu.async_copy(gs_hbm.at[pl.ds(base, CHUNK)], vals_dst, vsem)

    @pl.when(is_last)
    def _():
        pltpu.async_copy(gs_hbm.at[pl.ds(base, CHUNK_LAST)],
                         vals_dst.at[pl.ds(0, CHUNK_LAST)], vsem)

    abase = sid * CPT
    pltpu.async_copy(g_hbm.at[pl.ds(sid * GROWS, GROWS)], gbuf, gsem)
    pltpu.async_copy(sig_hbm.at[pl.ds(abase, CPT)], sigb, isems[1])
    pltpu.async_copy(loc_hbm.at[pl.ds(abase, CPT)], locb, fsems[1])
    pltpu.make_async_copy(sig_hbm.at[pl.ds(abase, CPT)], sigb,
                          isems[1]).wait()
    pltpu.make_async_copy(loc_hbm.at[pl.ds(abase, CPT)], locb,
                          fsems[1]).wait()
    pltpu.make_async_copy(g_hbm.at[pl.ds(sid * GROWS, GROWS)], gbuf,
                          gsem).wait()

    def fuse_row(j, carry):
        svs = [sigb[pl.ds(j * 128 + k * 16, 16)] for k in range(8)]
        lvs = [locb[pl.ds(j * 128 + k * 16, 16)] for k in range(8)]
        gvs = [gbuf[j, pl.ds(k * 16, 16)] for k in range(8)]
        for k in range(8):
            fusedb[pl.ds(j * 128 + k * 16, 16)] = (
                svs[k] * gvs[k] + (1.0 - svs[k]) * lvs[k])
        return carry

    lax.fori_loop(0, GROWS, fuse_row, 0)
    pltpu.sync_copy(fusedb, fused_sh.at[pl.ds(abase, CPT)])

    pltpu.async_copy(idx_hbm.at[pl.ds(0, CAND_CH)], idxsb.at[0], isems[0])

    plsc.subcore_barrier()

    pltpu.async_copy(fused_sh.at[pl.ds(0, CAND_CH)], fuseds.at[0], fsems[0])

    @pl.when(jnp.logical_not(is_last))
    def _():
        pltpu.make_async_copy(gs_hbm.at[pl.ds(base, CHUNK)], vals_dst,
                              vsem).wait()

    @pl.when(is_last)
    def _():
        pltpu.make_async_copy(gs_hbm.at[pl.ds(base, CHUNK_LAST)],
                              vals_dst.at[pl.ds(0, CHUNK_LAST)], vsem).wait()

    for c in range(N_CH):
        s = c & 1
        if c + 1 < N_CH:
            ns = 1 - s
            pltpu.async_copy(idx_hbm.at[pl.ds((c + 1) * CAND_CH, CAND_CH)],
                             idxsb.at[ns], isems[ns])
            pltpu.async_copy(fused_sh.at[pl.ds((c + 1) * CAND_CH, CAND_CH)],
                             fuseds.at[ns], fsems[ns])
        pltpu.make_async_copy(idx_hbm.at[pl.ds(c * CAND_CH, CAND_CH)],
                              idxsb.at[s], isems[s]).wait()
        pltpu.make_async_copy(fused_sh.at[pl.ds(c * CAND_CH, CAND_CH)],
                              fuseds.at[s], fsems[s]).wait()

        def row_body(j, carry, s=s):
            ivs = [idxsb[s, pl.ds(j * 128 + k * 16, 16)] for k in range(8)]
            fvs = [fuseds[s, pl.ds(j * 128 + k * 16, 16)] for k in range(8)]
            rels = [plsc.bitcast(iv - base, jnp.uint32) for iv in ivs]
            ms = [r < cwu for r in rels]
            relcs = [plsc.bitcast(jnp.minimum(r, jnp.uint32(CHUNK - 1)),
                                  jnp.int32) for r in rels]
            for k in range(8):
                plsc.store_scatter(vals_dst, [relcs[k]], fvs[k], mask=ms[k])
            return carry

        lax.fori_loop(0, ROWS_CH, row_body, 0)

    @pl.when(jnp.logical_not(is_last))
    def _():
        pltpu.sync_copy(vals_dst, out_hbm.at[pl.ds(base, CHUNK)])

    @pl.when(is_last)
    def _():
        pltpu.sync_copy(vals_dst.at[pl.ds(0, CHUNK_LAST)],
                        out_hbm.at[pl.ds(base, CHUNK_LAST)])


_sc_scatter = functools.partial(
    pl.kernel,
    out_type=jax.ShapeDtypeStruct((NUM_NODES,), jnp.float32),
    mesh=plsc.VectorSubcoreMesh(core_axis_name="c", subcore_axis_name="s",
                                num_cores=NC, num_subcores=NS),
    scratch_types=[
        pltpu.VMEM_SHARED((NUM_CAND,), jnp.float32),
        pltpu.VMEM((CHUNK,), jnp.float32),
        pltpu.VMEM((CPT,), jnp.float32),
        pltpu.VMEM((CPT,), jnp.float32),
        pltpu.VMEM((GROWS, 128), jnp.float32),
        pltpu.VMEM((CPT,), jnp.float32),
        pltpu.VMEM((2, CAND_CH), jnp.int32),
        pltpu.VMEM((2, CAND_CH), jnp.float32),
        pltpu.SemaphoreType.DMA,
        pltpu.SemaphoreType.DMA,
        pltpu.SemaphoreType.DMA,
        pltpu.SemaphoreType.DMA,
        pltpu.SemaphoreType.DMA,
        pltpu.SemaphoreType.DMA,
    ],
    compiler_params=pltpu.CompilerParams(needs_layout_passes=False),
)(_sc_body)


def kernel(global_emb, local_emb, global_scores, local_scores,
           candidate_indices, W1, b1, W2, b2):
    idx = candidate_indices.astype(jnp.int32)
    g2d = _sc_gather(global_scores, idx)
    sigma = _compute_sigma(global_emb, local_emb, W1, b1, W2, b2)
    fused = _sc_scatter(global_scores, idx, sigma, local_scores, g2d)
    return (fused, sigma)


# speedup vs baseline: 1.8491x; 1.0879x over previous
import functools

import jax
import jax.numpy as jnp
from jax import lax
from jax.experimental import pallas as pl
from jax.experimental.pallas import tpu as pltpu
from jax.experimental.pallas import tpu_sc as plsc

NUM_NODES = 1000000
NUM_CAND = 65536
GLOBAL_DIM = 128
LOCAL_DIM = 128
HID = 32

NC = 2
NS = 16
NW = NC * NS

CHUNK = 31256
CHUNK_LAST = NUM_NODES - (NW - 1) * CHUNK

CAND_CH = 8192
N_CH = NUM_CAND // CAND_CH

BR = 16384
GRID = NUM_CAND // BR


def _ffn_body(g_ref, l_ref, w1a_ref, w1b_ref, b1_ref, w2_ref, b2_ref, o_ref):
    dn = (((0,), (1,)), ((), ()))
    ht = lax.dot_general(w1a_ref[...], g_ref[...], dn,
                         preferred_element_type=jnp.float32)
    ht = ht + lax.dot_general(w1b_ref[...], l_ref[...], dn,
                              preferred_element_type=jnp.float32)
    ht = jnp.maximum(ht + b1_ref[...], 0.0)
    st = lax.dot_general(w2_ref[...], ht, (((0,), (0,)), ((), ())),
                         preferred_element_type=jnp.float32)
    o_ref[...] = jax.nn.sigmoid(st + b2_ref[...]).reshape(1, 1, BR)


def _compute_sigma(global_emb, local_emb, W1, b1, W2, b2):
    w1a = W1[:GLOBAL_DIM]
    w1b = W1[GLOBAL_DIM:]
    b1c = b1.reshape(HID, 1)
    b2c = b2.reshape(1, 1)
    out = pl.pallas_call(
        _ffn_body,
        grid=(GRID,),
        in_specs=[
            pl.BlockSpec((BR, GLOBAL_DIM), lambda i: (i, 0)),
            pl.BlockSpec((BR, LOCAL_DIM), lambda i: (i, 0)),
            pl.BlockSpec((GLOBAL_DIM, HID), lambda i: (0, 0)),
            pl.BlockSpec((LOCAL_DIM, HID), lambda i: (0, 0)),
            pl.BlockSpec((HID, 1), lambda i: (0, 0)),
            pl.BlockSpec((HID, 1), lambda i: (0, 0)),
            pl.BlockSpec((1, 1), lambda i: (0, 0)),
        ],
        out_specs=pl.BlockSpec((1, 1, BR), lambda i: (i, 0, 0)),
        out_shape=jax.ShapeDtypeStruct((GRID, 1, BR), jnp.float32),
    )(global_emb, local_emb, w1a, w1b, b1c, W2, b2c)
    return out.reshape(NUM_CAND)


CPT = NUM_CAND // NS
GROWS = CPT // 128
ROWS_CH = CAND_CH // 128

GPT = NUM_CAND // NW
GPROWS = GPT // 128


def _sc_gather_body(gs_hbm, idx_hbm, g_hbm, idxg, gbuf, gsem):
    cid = lax.axis_index("c")
    sid = lax.axis_index("s")
    wid = sid * NC + cid
    gbase = wid * GPT
    pltpu.sync_copy(idx_hbm.at[pl.ds(gbase, GPT)], idxg)

    @pl.loop(0, GPROWS)
    def _(j):
        pltpu.async_copy(gs_hbm.at[idxg.at[pl.ds(j * 128, 128)]],
                         gbuf.at[j], gsem)

    @pl.loop(0, GPROWS)
    def _(j):
        pltpu.make_async_copy(gs_hbm.at[idxg.at[pl.ds(j * 128, 128)]],
                              gbuf.at[j], gsem).wait()

    pltpu.sync_copy(gbuf, g_hbm.at[pl.ds(wid * GPROWS, GPROWS)])


_sc_gather = functools.partial(
    pl.kernel,
    out_type=jax.ShapeDtypeStruct((NUM_CAND // 128, 128), jnp.float32),
    mesh=plsc.VectorSubcoreMesh(core_axis_name="c", subcore_axis_name="s",
                                num_cores=NC, num_subcores=NS),
    scratch_types=[
        pltpu.VMEM((GPT,), jnp.int32),
        pltpu.VMEM((GPROWS, 128), jnp.float32),
        pltpu.SemaphoreType.DMA,
    ],
    compiler_params=pltpu.CompilerParams(needs_layout_passes=False),
)(_sc_gather_body)


def _sc_body(gs_hbm, idx_hbm, sig_hbm, loc_hbm, g_hbm, out_hbm,
             fused_sh, vals_dst, sigb, locb, gbuf, fusedb,
             idxsb, fuseds, gsem, vsem, isem0, isem1, fsem0, fsem1):
    cid = lax.axis_index("c")
    sid = lax.axis_index("s")
    wid = sid * NC + cid
    base = wid * CHUNK
    is_last = wid == NW - 1
    cw = jnp.where(is_last, CHUNK_LAST, CHUNK).astype(jnp.int32)
    cwu = cw.astype(jnp.uint32)
    isems = (isem0, isem1)
    fsems = (fsem0, fsem1)

    @pl.when(jnp.logical_not(is_last))
    def _():
        pltpu.async_copy(gs_hbm.at[pl.ds(base, CHUNK)], vals_dst, vsem)

    @pl.when(is_last)
    def _():
        pltpu.async_copy(gs_hbm.at[pl.ds(base, CHUNK_LAST)],
                         vals_dst.at[pl.ds(0, CHUNK_LAST)], vsem)

    abase = sid * CPT
    pltpu.async_copy(g_hbm.at[pl.ds(sid * GROWS, GROWS)], gbuf, gsem)
    pltpu.async_copy(sig_hbm.at[pl.ds(abase, CPT)], sigb, isems[1])
    pltpu.async_copy(loc_hbm.at[pl.ds(abase, CPT)], locb, fsems[1])
    pltpu.make_async_copy(sig_hbm.at[pl.ds(abase, CPT)], sigb,
                          isems[1]).wait()
    pltpu.make_async_copy(loc_hbm.at[pl.ds(abase, CPT)], locb,
                          fsems[1]).wait()
    pltpu.make_async_copy(g_hbm.at[pl.ds(sid * GROWS, GROWS)], gbuf,
                          gsem).wait()

    def fuse_row(j, carry):
        svs = [sigb[pl.ds(j * 128 + k * 16, 16)] for k in range(8)]
        lvs = [locb[pl.ds(j * 128 + k * 16, 16)] for k in range(8)]
        gvs = [gbuf[j, pl.ds(k * 16, 16)] for k in range(8)]
        for k in range(8):
            fusedb[pl.ds(j * 128 + k * 16, 16)] = (
                svs[k] * gvs[k] + (1.0 - svs[k]) * lvs[k])
        return carry

    lax.fori_loop(0, GROWS, fuse_row, 0)
    pltpu.sync_copy(fusedb, fused_sh.at[pl.ds(abase, CPT)])

    pltpu.async_copy(idx_hbm.at[pl.ds(0, CAND_CH)], idxsb.at[0], isems[0])

    plsc.subcore_barrier()

    pltpu.async_copy(fused_sh.at[pl.ds(0, CAND_CH)], fuseds.at[0], fsems[0])

    @pl.when(jnp.logical_not(is_last))
    def _():
        pltpu.make_async_copy(gs_hbm.at[pl.ds(base, CHUNK)], vals_dst,
                              vsem).wait()

    @pl.when(is_last)
    def _():
        pltpu.make_async_copy(gs_hbm.at[pl.ds(base, CHUNK_LAST)],
                              vals_dst.at[pl.ds(0, CHUNK_LAST)], vsem).wait()

    for c in range(N_CH):
        s = c & 1
        if c + 1 < N_CH:
            ns = 1 - s
            pltpu.async_copy(idx_hbm.at[pl.ds((c + 1) * CAND_CH, CAND_CH)],
                             idxsb.at[ns], isems[ns])
            pltpu.async_copy(fused_sh.at[pl.ds((c + 1) * CAND_CH, CAND_CH)],
                             fuseds.at[ns], fsems[ns])
        pltpu.make_async_copy(idx_hbm.at[pl.ds(c * CAND_CH, CAND_CH)],
                              idxsb.at[s], isems[s]).wait()
        pltpu.make_async_copy(fused_sh.at[pl.ds(c * CAND_CH, CAND_CH)],
                              fuseds.at[s], fsems[s]).wait()

        def row_body(j, carry, s=s):
            ivs = [idxsb[s, pl.ds(j * 128 + k * 16, 16)] for k in range(8)]
            fvs = [fuseds[s, pl.ds(j * 128 + k * 16, 16)] for k in range(8)]
            rels = [plsc.bitcast(iv - base, jnp.uint32) for iv in ivs]
            ms = [r < cwu for r in rels]
            relcs = [plsc.bitcast(jnp.minimum(r, jnp.uint32(CHUNK - 1)),
                                  jnp.int32) for r in rels]
            for k in range(8):
                plsc.store_scatter(vals_dst, [relcs[k]], fvs[k], mask=ms[k])
            return carry

        lax.fori_loop(0, ROWS_CH, row_body, 0)

    @pl.when(jnp.logical_not(is_last))
    def _():
        pltpu.sync_copy(vals_dst, out_hbm.at[pl.ds(base, CHUNK)])

    @pl.when(is_last)
    def _():
        pltpu.sync_copy(vals_dst.at[pl.ds(0, CHUNK_LAST)],
                        out_hbm.at[pl.ds(base, CHUNK_LAST)])


_sc_scatter = functools.partial(
    pl.kernel,
    out_type=jax.ShapeDtypeStruct((NUM_NODES,), jnp.float32),
    mesh=plsc.VectorSubcoreMesh(core_axis_name="c", subcore_axis_name="s",
                                num_cores=NC, num_subcores=NS),
    scratch_types=[
        pltpu.VMEM_SHARED((NUM_CAND,), jnp.float32),
        pltpu.VMEM((CHUNK,), jnp.float32),
        pltpu.VMEM((CPT,), jnp.float32),
        pltpu.VMEM((CPT,), jnp.float32),
        pltpu.VMEM((GROWS, 128), jnp.float32),
        pltpu.VMEM((CPT,), jnp.float32),
        pltpu.VMEM((2, CAND_CH), jnp.int32),
        pltpu.VMEM((2, CAND_CH), jnp.float32),
        pltpu.SemaphoreType.DMA,
        pltpu.SemaphoreType.DMA,
        pltpu.SemaphoreType.DMA,
        pltpu.SemaphoreType.DMA,
        pltpu.SemaphoreType.DMA,
        pltpu.SemaphoreType.DMA,
    ],
    compiler_params=pltpu.CompilerParams(needs_layout_passes=False),
)(_sc_body)


def kernel(global_emb, local_emb, global_scores, local_scores,
           candidate_indices, W1, b1, W2, b2):
    idx = candidate_indices.astype(jnp.int32)
    g2d = _sc_gather(global_scores, idx)
    sigma = _compute_sigma(global_emb, local_emb, W1, b1, W2, b2)
    fused = _sc_scatter(global_scores, idx, sigma, local_scores, g2d)
    return (fused, sigma)


# speedup vs baseline: 1.8930x; 1.0237x over previous
import functools

import jax
import jax.numpy as jnp
from jax import lax
from jax.experimental import pallas as pl
from jax.experimental.pallas import tpu as pltpu
from jax.experimental.pallas import tpu_sc as plsc

NUM_NODES = 1000000
NUM_CAND = 65536
GLOBAL_DIM = 128
LOCAL_DIM = 128
HID = 32

NC = 2
NS = 16
NW = NC * NS

CHUNK = 31256
CHUNK_LAST = NUM_NODES - (NW - 1) * CHUNK

CAND_CH = 16384
N_CH = NUM_CAND // CAND_CH

BR = 16384
GRID = NUM_CAND // BR


def _ffn_body(g_ref, l_ref, w1a_ref, w1b_ref, b1_ref, w2_ref, b2_ref, o_ref):
    dn = (((0,), (1,)), ((), ()))
    ht = lax.dot_general(w1a_ref[...], g_ref[...], dn,
                         preferred_element_type=jnp.float32)
    ht = ht + lax.dot_general(w1b_ref[...], l_ref[...], dn,
                              preferred_element_type=jnp.float32)
    ht = jnp.maximum(ht + b1_ref[...], 0.0)
    st = lax.dot_general(w2_ref[...], ht, (((0,), (0,)), ((), ())),
                         preferred_element_type=jnp.float32)
    o_ref[...] = jax.nn.sigmoid(st + b2_ref[...]).reshape(1, 1, BR)


def _compute_sigma(global_emb, local_emb, W1, b1, W2, b2):
    w1a = W1[:GLOBAL_DIM]
    w1b = W1[GLOBAL_DIM:]
    b1c = b1.reshape(HID, 1)
    b2c = b2.reshape(1, 1)
    out = pl.pallas_call(
        _ffn_body,
        grid=(GRID,),
        in_specs=[
            pl.BlockSpec((BR, GLOBAL_DIM), lambda i: (i, 0)),
            pl.BlockSpec((BR, LOCAL_DIM), lambda i: (i, 0)),
            pl.BlockSpec((GLOBAL_DIM, HID), lambda i: (0, 0)),
            pl.BlockSpec((LOCAL_DIM, HID), lambda i: (0, 0)),
            pl.BlockSpec((HID, 1), lambda i: (0, 0)),
            pl.BlockSpec((HID, 1), lambda i: (0, 0)),
            pl.BlockSpec((1, 1), lambda i: (0, 0)),
        ],
        out_specs=pl.BlockSpec((1, 1, BR), lambda i: (i, 0, 0)),
        out_shape=jax.ShapeDtypeStruct((GRID, 1, BR), jnp.float32),
    )(global_emb, local_emb, w1a, w1b, b1c, W2, b2c)
    return out.reshape(NUM_CAND)


CPT = NUM_CAND // NS
GROWS = CPT // 128
ROWS_CH = CAND_CH // 128

GPT = NUM_CAND // NW
GPROWS = GPT // 128


def _sc_gather_body(gs_hbm, idx_hbm, g_hbm, idxg, gbuf, gsem):
    cid = lax.axis_index("c")
    sid = lax.axis_index("s")
    wid = sid * NC + cid
    gbase = wid * GPT
    pltpu.sync_copy(idx_hbm.at[pl.ds(gbase, GPT)], idxg)

    @pl.loop(0, GPROWS)
    def _(j):
        pltpu.async_copy(gs_hbm.at[idxg.at[pl.ds(j * 128, 128)]],
                         gbuf.at[j], gsem)

    @pl.loop(0, GPROWS)
    def _(j):
        pltpu.make_async_copy(gs_hbm.at[idxg.at[pl.ds(j * 128, 128)]],
                              gbuf.at[j], gsem).wait()

    pltpu.sync_copy(gbuf, g_hbm.at[pl.ds(wid * GPROWS, GPROWS)])


_sc_gather = functools.partial(
    pl.kernel,
    out_type=jax.ShapeDtypeStruct((NUM_CAND // 128, 128), jnp.float32),
    mesh=plsc.VectorSubcoreMesh(core_axis_name="c", subcore_axis_name="s",
                                num_cores=NC, num_subcores=NS),
    scratch_types=[
        pltpu.VMEM((GPT,), jnp.int32),
        pltpu.VMEM((GPROWS, 128), jnp.float32),
        pltpu.SemaphoreType.DMA,
    ],
    compiler_params=pltpu.CompilerParams(needs_layout_passes=False),
)(_sc_gather_body)


def _sc_body(gs_hbm, idx_hbm, sig_hbm, loc_hbm, g_hbm, out_hbm,
             fused_sh, vals_dst, sigb, locb, gbuf, fusedb,
             idxsb, fuseds, gsem, vsem, isem0, isem1, fsem0, fsem1):
    cid = lax.axis_index("c")
    sid = lax.axis_index("s")
    wid = sid * NC + cid
    base = wid * CHUNK
    is_last = wid == NW - 1
    cw = jnp.where(is_last, CHUNK_LAST, CHUNK).astype(jnp.int32)
    cwu = cw.astype(jnp.uint32)
    isems = (isem0, isem1)
    fsems = (fsem0, fsem1)

    @pl.when(jnp.logical_not(is_last))
    def _():
        pltpu.async_copy(gs_hbm.at[pl.ds(base, CHUNK)], vals_dst, vsem)

    @pl.when(is_last)
    def _():
        pltpu.async_copy(gs_hbm.at[pl.ds(base, CHUNK_LAST)],
                         vals_dst.at[pl.ds(0, CHUNK_LAST)], vsem)

    abase = sid * CPT
    pltpu.async_copy(g_hbm.at[pl.ds(sid * GROWS, GROWS)], gbuf, gsem)
    pltpu.async_copy(sig_hbm.at[pl.ds(abase, CPT)], sigb, isems[1])
    pltpu.async_copy(loc_hbm.at[pl.ds(abase, CPT)], locb, fsems[1])
    pltpu.make_async_copy(sig_hbm.at[pl.ds(abase, CPT)], sigb,
                          isems[1]).wait()
    pltpu.make_async_copy(loc_hbm.at[pl.ds(abase, CPT)], locb,
                          fsems[1]).wait()
    pltpu.make_async_copy(g_hbm.at[pl.ds(sid * GROWS, GROWS)], gbuf,
                          gsem).wait()

    def fuse_row(j, carry):
        svs = [sigb[pl.ds(j * 128 + k * 16, 16)] for k in range(8)]
        lvs = [locb[pl.ds(j * 128 + k * 16, 16)] for k in range(8)]
        gvs = [gbuf[j, pl.ds(k * 16, 16)] for k in range(8)]
        for k in range(8):
            fusedb[pl.ds(j * 128 + k * 16, 16)] = (
                svs[k] * gvs[k] + (1.0 - svs[k]) * lvs[k])
        return carry

    lax.fori_loop(0, GROWS, fuse_row, 0)
    pltpu.sync_copy(fusedb, fused_sh.at[pl.ds(abase, CPT)])

    pltpu.async_copy(idx_hbm.at[pl.ds(0, CAND_CH)], idxsb.at[0], isems[0])

    plsc.subcore_barrier()

    pltpu.async_copy(fused_sh.at[pl.ds(0, CAND_CH)], fuseds.at[0], fsems[0])

    @pl.when(jnp.logical_not(is_last))
    def _():
        pltpu.make_async_copy(gs_hbm.at[pl.ds(base, CHUNK)], vals_dst,
                              vsem).wait()

    @pl.when(is_last)
    def _():
        pltpu.make_async_copy(gs_hbm.at[pl.ds(base, CHUNK_LAST)],
                              vals_dst.at[pl.ds(0, CHUNK_LAST)], vsem).wait()

    for c in range(N_CH):
        s = c & 1
        if c + 1 < N_CH:
            ns = 1 - s
            pltpu.async_copy(idx_hbm.at[pl.ds((c + 1) * CAND_CH, CAND_CH)],
                             idxsb.at[ns], isems[ns])
            pltpu.async_copy(fused_sh.at[pl.ds((c + 1) * CAND_CH, CAND_CH)],
                             fuseds.at[ns], fsems[ns])
        pltpu.make_async_copy(idx_hbm.at[pl.ds(c * CAND_CH, CAND_CH)],
                              idxsb.at[s], isems[s]).wait()
        pltpu.make_async_copy(fused_sh.at[pl.ds(c * CAND_CH, CAND_CH)],
                              fuseds.at[s], fsems[s]).wait()

        def row_body(j, carry, s=s):
            ivs = [idxsb[s, pl.ds(j * 128 + k * 16, 16)] for k in range(8)]
            fvs = [fuseds[s, pl.ds(j * 128 + k * 16, 16)] for k in range(8)]
            rels = [plsc.bitcast(iv - base, jnp.uint32) for iv in ivs]
            ms = [r < cwu for r in rels]
            relcs = [plsc.bitcast(jnp.minimum(r, jnp.uint32(CHUNK - 1)),
                                  jnp.int32) for r in rels]
            for k in range(8):
                plsc.store_scatter(vals_dst, [relcs[k]], fvs[k], mask=ms[k])
            return carry

        lax.fori_loop(0, ROWS_CH, row_body, 0)

    @pl.when(jnp.logical_not(is_last))
    def _():
        pltpu.sync_copy(vals_dst, out_hbm.at[pl.ds(base, CHUNK)])

    @pl.when(is_last)
    def _():
        pltpu.sync_copy(vals_dst.at[pl.ds(0, CHUNK_LAST)],
                        out_hbm.at[pl.ds(base, CHUNK_LAST)])


_sc_scatter = functools.partial(
    pl.kernel,
    out_type=jax.ShapeDtypeStruct((NUM_NODES,), jnp.float32),
    mesh=plsc.VectorSubcoreMesh(core_axis_name="c", subcore_axis_name="s",
                                num_cores=NC, num_subcores=NS),
    scratch_types=[
        pltpu.VMEM_SHARED((NUM_CAND,), jnp.float32),
        pltpu.VMEM((CHUNK,), jnp.float32),
        pltpu.VMEM((CPT,), jnp.float32),
        pltpu.VMEM((CPT,), jnp.float32),
        pltpu.VMEM((GROWS, 128), jnp.float32),
        pltpu.VMEM((CPT,), jnp.float32),
        pltpu.VMEM((2, CAND_CH), jnp.int32),
        pltpu.VMEM((2, CAND_CH), jnp.float32),
        pltpu.SemaphoreType.DMA,
        pltpu.SemaphoreType.DMA,
        pltpu.SemaphoreType.DMA,
        pltpu.SemaphoreType.DMA,
        pltpu.SemaphoreType.DMA,
        pltpu.SemaphoreType.DMA,
    ],
    compiler_params=pltpu.CompilerParams(needs_layout_passes=False),
)(_sc_body)


def kernel(global_emb, local_emb, global_scores, local_scores,
           candidate_indices, W1, b1, W2, b2):
    idx = candidate_indices.astype(jnp.int32)
    g2d = _sc_gather(global_scores, idx)
    sigma = _compute_sigma(global_emb, local_emb, W1, b1, W2, b2)
    fused = _sc_scatter(global_scores, idx, sigma, local_scores, g2d)
    return (fused, sigma)
